# baseline (speedup 1.0000x reference)
import functools

import jax
import jax.numpy as jnp
from jax.experimental import pallas as pl
from jax.experimental.pallas import tpu as pltpu

_BS, _NN, _D = 2, 512, 128
_NUM_REL = 2
_K = 2
_CH = _NN // _K

_CONTRACT0 = (((0,), (0,)), ((), ()))


def _split(v):
    vh = v.astype(jnp.bfloat16)
    vl = (v - vh.astype(jnp.float32)).astype(jnp.bfloat16)
    return vh, vl


def _mm3(xh, xl, wh, wl):
    return (jnp.dot(xh, wh, preferred_element_type=jnp.float32)
            + jnp.dot(xh, wl, preferred_element_type=jnp.float32)
            + jnp.dot(xl, wh, preferred_element_type=jnp.float32))


def _agg(a, yh):
    return jax.lax.dot_general(a, yh, _CONTRACT0,
                               preferred_element_type=jnp.float32)


def _masks(adj_ref):
    aug = adj_ref[0, 0]
    pun = adj_ref[1, 0]
    m1 = aug == 1
    m0 = (pun == 1) & (aug != 1)
    a1 = m1.astype(jnp.bfloat16)
    a0 = m0.astype(jnp.bfloat16)
    c0 = jnp.sum(m0.astype(jnp.float32), axis=0).reshape(1, _NN)
    c1 = jnp.sum(m1.astype(jnp.float32), axis=0).reshape(1, _NN)
    return a0, a1, c0, c1


def _rgcn_kernel(adj_ref, x_ref, wrel0_ref, wroot0_ref, b0_ref,
                 wrel1_ref, wroot1_ref, b1_ref, out_ref,
                 a0s, a1s, hroot_s, y0s, y1s, s0s, s1s, c0s, c1s):
    k = pl.program_id(1)
    a0c, a1c, c0c, c1c = _masks(adj_ref)
    a0s[k] = a0c
    a1s[k] = a1c

    @pl.when(k == 0)
    def _first_chunk():
        x = x_ref[0]
        xh, xl = _split(x)
        hroot_s[...] = _mm3(xh, xl, *_split(wroot0_ref[...])) + b0_ref[...]
        y0h = jnp.dot(xh, wrel0_ref[0].astype(jnp.bfloat16),
                      preferred_element_type=jnp.float32).astype(jnp.bfloat16)
        y1h = jnp.dot(xh, wrel0_ref[1].astype(jnp.bfloat16),
                      preferred_element_type=jnp.float32).astype(jnp.bfloat16)
        y0s[...] = y0h.reshape(_K, _CH, _D)
        y1s[...] = y1h.reshape(_K, _CH, _D)
        s0s[...] = _agg(a0c, y0h[:_CH])
        s1s[...] = _agg(a1c, y1h[:_CH])
        c0s[...] = c0c
        c1s[...] = c1c

    @pl.when(k > 0)
    def _later_chunk():
        s0s[...] += _agg(a0c, y0s[k])
        s1s[...] += _agg(a1c, y1s[k])
        c0s[...] += c0c
        c1s[...] += c1c

    @pl.when(k == _K - 1)
    def _finish():
        inv0 = (1.0 / jnp.maximum(c0s[...], 1.0)).reshape(_NN, 1)
        inv1 = (1.0 / jnp.maximum(c1s[...], 1.0)).reshape(_NN, 1)
        h = hroot_s[...] + s0s[...] * inv0 + s1s[...] * inv1
        x = jnp.where(h > 0, h, jnp.exp(jnp.minimum(h, 0.0)) - 1.0)

        xh, xl = _split(x)
        hroot = _mm3(xh, xl, *_split(wroot1_ref[...])) + b1_ref[...]
        y0h = jnp.dot(xh, wrel1_ref[0].astype(jnp.bfloat16),
                      preferred_element_type=jnp.float32).astype(jnp.bfloat16)
        y1h = jnp.dot(xh, wrel1_ref[1].astype(jnp.bfloat16),
                      preferred_element_type=jnp.float32).astype(jnp.bfloat16)
        a0full = a0s[...].reshape(_NN, _NN)
        a1full = a1s[...].reshape(_NN, _NN)
        h = (hroot + _agg(a0full, y0h) * inv0 + _agg(a1full, y1h) * inv1)
        out_ref[0] = jnp.where(h > 0, h, jnp.exp(jnp.minimum(h, 0.0)) - 1.0)


@functools.partial(jax.jit, static_argnames=())
def _run(adj, x, wrel0, wroot0, b0, wrel1, wroot1, b1):
    return pl.pallas_call(
        _rgcn_kernel,
        grid=(_BS, _K),
        in_specs=[
            pl.BlockSpec((2, 1, _CH, _NN), lambda b, k: (0, b, k, 0)),
            pl.BlockSpec((1, _NN, _D), lambda b, k: (b, 0, 0)),
            pl.BlockSpec((_NUM_REL, _D, _D), lambda b, k: (0, 0, 0)),
            pl.BlockSpec((_D, _D), lambda b, k: (0, 0)),
            pl.BlockSpec((1, _D), lambda b, k: (0, 0)),
            pl.BlockSpec((_NUM_REL, _D, _D), lambda b, k: (0, 0, 0)),
            pl.BlockSpec((_D, _D), lambda b, k: (0, 0)),
            pl.BlockSpec((1, _D), lambda b, k: (0, 0)),
        ],
        out_specs=pl.BlockSpec((1, _NN, _D), lambda b, k: (b, 0, 0)),
        out_shape=jax.ShapeDtypeStruct((_BS, _NN, _D), jnp.float32),
        scratch_shapes=[
            pltpu.VMEM((_K, _CH, _NN), jnp.bfloat16),
            pltpu.VMEM((_K, _CH, _NN), jnp.bfloat16),
            pltpu.VMEM((_NN, _D), jnp.float32),
            pltpu.VMEM((_K, _CH, _D), jnp.bfloat16),
            pltpu.VMEM((_K, _CH, _D), jnp.bfloat16),
            pltpu.VMEM((_NN, _D), jnp.float32),
            pltpu.VMEM((_NN, _D), jnp.float32),
            pltpu.VMEM((1, _NN), jnp.float32),
            pltpu.VMEM((1, _NN), jnp.float32),
        ],
    )(adj, x, wrel0, wroot0, b0, wrel1, wroot1, b1)


def kernel(feature_list, adj_list, aug_pun_adj, pooled_output, p_nodes_mask,
           o_nodes_mask, W_rel0, W_root0, bias0, W_rel1, W_root1, bias1):
    x = feature_list[0]
    adj = aug_pun_adj.astype(jnp.int32)
    out = _run(adj, x, W_rel0, W_root0, bias0.reshape(1, _D),
               W_rel1, W_root1, bias1.reshape(1, _D))
    return out


# speedup vs baseline: 1.3469x; 1.3469x over previous
import functools

import jax
import jax.numpy as jnp
from jax.experimental import pallas as pl

_BS, _NN, _D = 2, 512, 128
_NUM_REL = 2

_CONTRACT0 = (((0,), (0,)), ((), ()))


def _split(v):
    vh = v.astype(jnp.bfloat16)
    vl = (v - vh.astype(jnp.float32)).astype(jnp.bfloat16)
    return vh, vl


def _mm3(xh, xl, wh, wl):
    return (jnp.dot(xh, wh, preferred_element_type=jnp.float32)
            + jnp.dot(xh, wl, preferred_element_type=jnp.float32)
            + jnp.dot(xl, wh, preferred_element_type=jnp.float32))


def _mm1(xh, wh):
    return jnp.dot(xh, wh, preferred_element_type=jnp.float32)


def _agg(a, yh):
    return jax.lax.dot_general(a, yh, _CONTRACT0,
                               preferred_element_type=jnp.float32)


def _rgcn_kernel(adj_ref, x_ref, wrel0_ref, wroot0_ref, b0_ref,
                 wrel1_ref, wroot1_ref, b1_ref, out_ref):
    aug = adj_ref[0, 0]
    pun = adj_ref[1, 0]
    m1 = aug == 1
    m0 = (pun == 1) & (aug != 1)
    a1 = m1.astype(jnp.bfloat16)
    a0 = m0.astype(jnp.bfloat16)

    inv0 = 1.0 / jnp.maximum(jnp.sum(m0.astype(jnp.float32), axis=0), 1.0)
    inv1 = 1.0 / jnp.maximum(jnp.sum(m1.astype(jnp.float32), axis=0), 1.0)

    x = x_ref[0]
    for wrel_ref, wroot_ref, b_ref in ((wrel0_ref, wroot0_ref, b0_ref),
                                       (wrel1_ref, wroot1_ref, b1_ref)):
        wrh, wrl = _split(wroot_ref[...])
        w0h = wrel_ref[0].astype(jnp.bfloat16)
        w1h = wrel_ref[1].astype(jnp.bfloat16)
        xh = x.astype(jnp.bfloat16)
        hroot = (jnp.dot(xh, wrh, preferred_element_type=jnp.float32)
                 + jnp.dot(xh, wrl, preferred_element_type=jnp.float32)
                 + b_ref[...])
        y0h = _mm1(xh, w0h).astype(jnp.bfloat16)
        y1h = _mm1(xh, w1h).astype(jnp.bfloat16)
        h = (hroot + _agg(a0, y0h) * inv0[:, None]
             + _agg(a1, y1h) * inv1[:, None])
        x = jnp.where(h > 0, h, jnp.exp(jnp.minimum(h, 0.0)) - 1.0)
    out_ref[0] = x


@functools.partial(jax.jit, static_argnames=())
def _run(adj, x, wrel0, wroot0, b0, wrel1, wroot1, b1):
    return pl.pallas_call(
        _rgcn_kernel,
        grid=(_BS,),
        in_specs=[
            pl.BlockSpec((2, 1, _NN, _NN), lambda b: (0, b, 0, 0)),
            pl.BlockSpec((1, _NN, _D), lambda b: (b, 0, 0)),
            pl.BlockSpec((_NUM_REL, _D, _D), lambda b: (0, 0, 0)),
            pl.BlockSpec((_D, _D), lambda b: (0, 0)),
            pl.BlockSpec((1, _D), lambda b: (0, 0)),
            pl.BlockSpec((_NUM_REL, _D, _D), lambda b: (0, 0, 0)),
            pl.BlockSpec((_D, _D), lambda b: (0, 0)),
            pl.BlockSpec((1, _D), lambda b: (0, 0)),
        ],
        out_specs=pl.BlockSpec((1, _NN, _D), lambda b: (b, 0, 0)),
        out_shape=jax.ShapeDtypeStruct((_BS, _NN, _D), jnp.float32),
    )(adj, x, wrel0, wroot0, b0, wrel1, wroot1, b1)


def kernel(feature_list, adj_list, aug_pun_adj, pooled_output, p_nodes_mask,
           o_nodes_mask, W_rel0, W_root0, bias0, W_rel1, W_root1, bias1):
    x = feature_list[0]
    adj = aug_pun_adj.astype(jnp.int32)
    out = _run(adj, x, W_rel0, W_root0, bias0.reshape(1, _D),
               W_rel1, W_root1, bias1.reshape(1, _D))
    return out


# speedup vs baseline: 1.3573x; 1.0077x over previous
import functools

import jax
import jax.numpy as jnp
from jax.experimental import pallas as pl

_BS, _NN, _D = 2, 512, 128
_NUM_REL = 2

_CONTRACT0 = (((0,), (0,)), ((), ()))


def _split(v):
    vh = v.astype(jnp.bfloat16)
    vl = (v - vh.astype(jnp.float32)).astype(jnp.bfloat16)
    return vh, vl


def _mm1(xh, wh):
    return jnp.dot(xh, wh, preferred_element_type=jnp.float32)


def _agg(a, yh):
    return jax.lax.dot_general(a, yh, _CONTRACT0,
                               preferred_element_type=jnp.float32)


def _rgcn_kernel(adj_ref, x_ref, wrel0_ref, wroot0_ref, b0_ref,
                 wrel1_ref, wroot1_ref, b1_ref, out_ref):
    aug = adj_ref[0, 0]
    pun = adj_ref[1, 0]
    m1 = aug == 1
    m0 = (pun == 1) & (aug != 1)
    a1 = m1.astype(jnp.bfloat16)
    a0 = m0.astype(jnp.bfloat16)

    inv0 = 1.0 / jnp.maximum(jnp.sum(m0.astype(jnp.float32), axis=0), 1.0)
    inv1 = 1.0 / jnp.maximum(jnp.sum(m1.astype(jnp.float32), axis=0), 1.0)

    x = x_ref[0]
    for wrel_ref, wroot_ref, b_ref in ((wrel0_ref, wroot0_ref, b0_ref),
                                       (wrel1_ref, wroot1_ref, b1_ref)):
        wrh, wrl = _split(wroot_ref[...])
        w0h = wrel_ref[0].astype(jnp.bfloat16)
        w1h = wrel_ref[1].astype(jnp.bfloat16)
        xh = x.astype(jnp.bfloat16)
        hroot = (jnp.dot(xh, wrh, preferred_element_type=jnp.float32)
                 + jnp.dot(xh, wrl, preferred_element_type=jnp.float32)
                 + b_ref[...])
        y0h = _mm1(xh, w0h).astype(jnp.bfloat16)
        y1h = _mm1(xh, w1h).astype(jnp.bfloat16)
        h = (hroot + _agg(a0, y0h) * inv0[:, None]
             + _agg(a1, y1h) * inv1[:, None])
        x = jnp.where(h > 0, h, jnp.exp(jnp.minimum(h, 0.0)) - 1.0)
    out_ref[0] = x


@functools.partial(jax.jit, static_argnames=())
def _run(adj, x, wrel0, wroot0, b0, wrel1, wroot1, b1):
    return pl.pallas_call(
        _rgcn_kernel,
        grid=(_BS,),
        in_specs=[
            pl.BlockSpec((2, 1, _NN, _NN), lambda b: (0, b, 0, 0)),
            pl.BlockSpec((1, _NN, _D), lambda b: (b, 0, 0)),
            pl.BlockSpec((_NUM_REL, _D, _D), lambda b: (0, 0, 0)),
            pl.BlockSpec((_D, _D), lambda b: (0, 0)),
            pl.BlockSpec((1, _D), lambda b: (0, 0)),
            pl.BlockSpec((_NUM_REL, _D, _D), lambda b: (0, 0, 0)),
            pl.BlockSpec((_D, _D), lambda b: (0, 0)),
            pl.BlockSpec((1, _D), lambda b: (0, 0)),
        ],
        out_specs=pl.BlockSpec((1, _NN, _D), lambda b: (b, 0, 0)),
        out_shape=jax.ShapeDtypeStruct((_BS, _NN, _D), jnp.float32),
    )(adj, x, wrel0, wroot0, b0, wrel1, wroot1, b1)


def kernel(feature_list, adj_list, aug_pun_adj, pooled_output, p_nodes_mask,
           o_nodes_mask, W_rel0, W_root0, bias0, W_rel1, W_root1, bias1):
    x = feature_list[0]
    adj = aug_pun_adj.astype(jnp.int32)
    out = _run(adj, x, W_rel0, W_root0, bias0.reshape(1, _D),
               W_rel1, W_root1, bias1.reshape(1, _D))
    return out
